# baseline (speedup 1.0000x reference)
import functools

import jax
import jax.numpy as jnp
from jax import lax
from jax.experimental import pallas as pl
from jax.experimental.pallas import tpu as pltpu
from jax.experimental.pallas import tpu_sc as plsc

N_ATOMS = 10000
MAX_NEI = 32
HIDDEN = 128
ATOM_FDIM = 133
BOND_FDIM = 14
DEPTH = 3

NW = 32
APW = 320
N_PAD = NW * APW
CHUNK_ATOMS = 4
CHUNKS = APW // CHUNK_ATOMS


def _gelu_exact(x):
    return 0.5 * x * (1.0 + lax.erf(x * 0.7071067811865476))


def _make_sc_gather_sum(n_rows, width):
    groups = width // 16
    mesh = plsc.VectorSubcoreMesh(core_axis_name="c", subcore_axis_name="s")

    @functools.partial(
        pl.kernel,
        out_type=jax.ShapeDtypeStruct((N_PAD, width), jnp.float32),
        mesh=mesh,
        compiler_params=pltpu.CompilerParams(use_tc_tiling_on_sc=False),
        scratch_types=[
            pltpu.VMEM((CHUNKS, 128), jnp.int32),
            pltpu.VMEM((4, 128, width), jnp.float32),
            pltpu.VMEM((4, CHUNK_ATOMS, width), jnp.float32),
            pltpu.SemaphoreType.DMA,
            pltpu.SemaphoreType.DMA,
            pltpu.SemaphoreType.DMA,
            pltpu.SemaphoreType.DMA,
            pltpu.SemaphoreType.DMA,
            pltpu.SemaphoreType.DMA,
            pltpu.SemaphoreType.DMA,
            pltpu.SemaphoreType.DMA,
        ],
    )
    def gsum(table_hbm, idx_hbm, out_hbm, idx_v, rows_v, out_v,
             sem0, sem1, sem2, sem3, osem0, osem1, osem2, osem3):
        wid = lax.axis_index("s") * 2 + lax.axis_index("c")
        pltpu.sync_copy(idx_hbm.at[wid], idx_v)
        sems = (sem0, sem1, sem2, sem3)
        osems = (osem0, osem1, osem2, osem3)
        nbuf = 4

        def copy(c, b):
            return pltpu.make_async_copy(
                table_hbm.at[idx_v.at[c]], rows_v.at[b], sems[b])

        def out_copy(c, b):
            return pltpu.make_async_copy(
                out_v.at[b],
                out_hbm.at[pl.ds(wid * APW + c * CHUNK_ATOMS, CHUNK_ATOMS)],
                osems[b])

        for b in range(nbuf - 1):
            copy(b, b).start()

        def quad_body(i, _):
            c0 = i * nbuf
            for b in range(nbuf):
                c = c0 + b

                @pl.when(c + nbuf - 1 < CHUNKS)
                def _():
                    copy(c + nbuf - 1, (b + nbuf - 1) % nbuf).start()

                copy(c, b).wait()

                @pl.when(c >= nbuf)
                def _():
                    out_copy(c - nbuf, b).wait()

                for a in range(CHUNK_ATOMS):
                    accs = [rows_v[b, a * MAX_NEI, pl.ds(16 * g, 16)]
                            for g in range(groups)]
                    for r in range(1, MAX_NEI):
                        for g in range(groups):
                            accs[g] = accs[g] + rows_v[
                                b, a * MAX_NEI + r, pl.ds(16 * g, 16)]
                    for g in range(groups):
                        out_v[b, a, pl.ds(16 * g, 16)] = accs[g]
                out_copy(c, b).start()
            return 0

        lax.fori_loop(0, CHUNKS // nbuf, quad_body, 0)
        for b in range(nbuf):
            out_copy(CHUNKS - nbuf + b, b).wait()

    return gsum


_gsum_msg = _make_sc_gather_sum(N_ATOMS, HIDDEN)
_gsum_bond = _make_sc_gather_sum(320000, 16)


_ROWS = 2000
_GRID = N_ATOMS // _ROWS


def _row_mask(pid, x):
    rows = lax.broadcasted_iota(jnp.int32, x.shape, 0) + pid * _ROWS
    return jnp.where(rows == 0, 0.0, x)


def _prologue_body(x_ref, wi_ref, w0_ref, w1_ref, w2_ref, inp_ref, h_ref):
    pid = pl.program_id(0)
    x = x_ref[...]
    inp = jnp.dot(x, wi_ref[...], preferred_element_type=jnp.float32)
    inp_ref[...] = _row_mask(pid, inp)
    h = _gelu_exact(jnp.dot(x, w0_ref[...], preferred_element_type=jnp.float32))
    h = _gelu_exact(jnp.dot(h, w1_ref[...], preferred_element_type=jnp.float32))
    h = _gelu_exact(jnp.dot(h, w2_ref[...], preferred_element_type=jnp.float32))
    h_ref[...] = h


def _tc_prologue(f_atoms, W_i, W0, W1, W2):
    return pl.pallas_call(
        _prologue_body,
        grid=(_GRID,),
        in_specs=[
            pl.BlockSpec((_ROWS, ATOM_FDIM), lambda i: (i, 0)),
            pl.BlockSpec((ATOM_FDIM, HIDDEN), lambda i: (0, 0)),
            pl.BlockSpec((ATOM_FDIM, HIDDEN), lambda i: (0, 0)),
            pl.BlockSpec((HIDDEN, HIDDEN), lambda i: (0, 0)),
            pl.BlockSpec((HIDDEN, HIDDEN), lambda i: (0, 0)),
        ],
        out_specs=[
            pl.BlockSpec((_ROWS, HIDDEN), lambda i: (i, 0)),
            pl.BlockSpec((_ROWS, HIDDEN), lambda i: (i, 0)),
        ],
        out_shape=[
            jax.ShapeDtypeStruct((N_ATOMS, HIDDEN), jnp.float32),
            jax.ShapeDtypeStruct((N_ATOMS, HIDDEN), jnp.float32),
        ],
    )(f_atoms, W_i, W0, W1, W2)


def _update_body(m_ref, s_ref, b_ref, wt_ref, wb_ref, o_ref):
    pid = pl.program_id(0)
    m = (m_ref[...]
         + jnp.dot(s_ref[...], wt_ref[...], preferred_element_type=jnp.float32)
         + jnp.dot(b_ref[...], wb_ref[...], preferred_element_type=jnp.float32))
    o_ref[...] = _row_mask(pid, m)


def _tc_update(message, s, sumb, Wh_top, Wh_bot16):
    return pl.pallas_call(
        _update_body,
        grid=(_GRID,),
        in_specs=[
            pl.BlockSpec((_ROWS, HIDDEN), lambda i: (i, 0)),
            pl.BlockSpec((_ROWS, HIDDEN), lambda i: (i, 0)),
            pl.BlockSpec((_ROWS, 16), lambda i: (i, 0)),
            pl.BlockSpec((HIDDEN, HIDDEN), lambda i: (0, 0)),
            pl.BlockSpec((16, HIDDEN), lambda i: (0, 0)),
        ],
        out_specs=pl.BlockSpec((_ROWS, HIDDEN), lambda i: (i, 0)),
        out_shape=jax.ShapeDtypeStruct((N_ATOMS, HIDDEN), jnp.float32),
    )(message, s, sumb, Wh_top, Wh_bot16)


def _final_body(h_ref, s_ref, wt_ref, wb_ref, o_ref):
    o = (jnp.dot(h_ref[...], wt_ref[...], preferred_element_type=jnp.float32)
         + jnp.dot(s_ref[...], wb_ref[...], preferred_element_type=jnp.float32))
    o_ref[...] = _gelu_exact(o)


def _tc_final(h, s, Wo_top, Wo_bot):
    return pl.pallas_call(
        _final_body,
        grid=(_GRID,),
        in_specs=[
            pl.BlockSpec((_ROWS, HIDDEN), lambda i: (i, 0)),
            pl.BlockSpec((_ROWS, HIDDEN), lambda i: (i, 0)),
            pl.BlockSpec((HIDDEN, HIDDEN), lambda i: (0, 0)),
            pl.BlockSpec((HIDDEN, HIDDEN), lambda i: (0, 0)),
        ],
        out_specs=pl.BlockSpec((_ROWS, HIDDEN), lambda i: (i, 0)),
        out_shape=jax.ShapeDtypeStruct((N_ATOMS, HIDDEN), jnp.float32),
    )(h, s, Wo_top, Wo_bot)


def _pack_idx(idx):
    idx = jnp.pad(idx.astype(jnp.int32), ((0, N_PAD - N_ATOMS), (0, 0)))
    return idx.reshape(NW, CHUNKS, 128)


def kernel(f_atoms, f_bonds, a2a, a2b, W_i, W_ah0, W_ah1, W_ah2,
           W_h0, W_h1, W_h2, W_o):
    idx_a = _pack_idx(a2a)
    idx_b = _pack_idx(a2b)
    f_bonds16 = jnp.pad(f_bonds, ((0, 0), (0, 16 - BOND_FDIM)))

    W_h = [W_h0, W_h1, W_h2]
    Wh_top = [w[:HIDDEN] for w in W_h]
    Wh_bot16 = [jnp.pad(w[HIDDEN:], ((0, 2), (0, 0))) for w in W_h]

    inp, h = _tc_prologue(f_atoms, W_i, W_ah0, W_ah1, W_ah2)
    sumb = _gsum_bond(f_bonds16, idx_b)[:N_ATOMS]

    message = inp
    for d in range(DEPTH):
        s = _gsum_msg(message, idx_a)[:N_ATOMS]
        message = _tc_update(message, s, sumb, Wh_top[d], Wh_bot16[d])

    s = _gsum_msg(message, idx_a)[:N_ATOMS]
    return _tc_final(h, s, W_o[:HIDDEN], W_o[HIDDEN:])


# speedup vs baseline: 2.9165x; 2.9165x over previous
import functools

import jax
import jax.numpy as jnp
from jax import lax
from jax.experimental import pallas as pl
from jax.experimental.pallas import tpu as pltpu
from jax.experimental.pallas import tpu_sc as plsc

N_ATOMS = 10000
MAX_NEI = 32
HIDDEN = 128
ATOM_FDIM = 133
BOND_FDIM = 14
DEPTH = 3

NW = 32
APW = 320
N_PAD = NW * APW
CHUNK_ATOMS = 4
CHUNKS = APW // CHUNK_ATOMS


def _gelu_exact(x):
    return 0.5 * x * (1.0 + lax.erf(x * 0.7071067811865476))


def _make_sc_gather_sum(n_rows, width, dtype, stage_spmem):
    lanes = 32 if dtype == jnp.bfloat16 else 16
    groups = width // lanes
    mesh = plsc.VectorSubcoreMesh(core_axis_name="c", subcore_axis_name="s")
    scratch = [
        pltpu.VMEM((CHUNKS, 128), jnp.int32),
        pltpu.VMEM((4, 128, width), dtype),
        pltpu.VMEM((4, CHUNK_ATOMS, width), dtype),
    ] + [pltpu.SemaphoreType.DMA] * 8
    if stage_spmem:
        scratch.append(pltpu.VMEM_SHARED((n_rows, width), dtype))

    @functools.partial(
        pl.kernel,
        out_type=jax.ShapeDtypeStruct((N_PAD, width), dtype),
        mesh=mesh,
        compiler_params=pltpu.CompilerParams(use_tc_tiling_on_sc=False,
                                             needs_layout_passes=False),
        scratch_types=scratch,
    )
    def gsum(table_hbm, idx_hbm, out_hbm, idx_v, rows_v, out_v,
             sem0, sem1, sem2, sem3, osem0, osem1, osem2, osem3,
             *maybe_shared):
        wid = lax.axis_index("s") * 2 + lax.axis_index("c")
        pltpu.sync_copy(idx_hbm.at[wid], idx_v)
        sems = (sem0, sem1, sem2, sem3)
        osems = (osem0, osem1, osem2, osem3)
        nbuf = 4

        if stage_spmem:
            shared = maybe_shared[0]
            sub = lax.axis_index("s")
            rpw = n_rows // 16
            pltpu.sync_copy(table_hbm.at[pl.ds(sub * rpw, rpw)],
                            shared.at[pl.ds(sub * rpw, rpw)])
            plsc.subcore_barrier()
            src = shared
        else:
            src = table_hbm

        def copy(c, b):
            return pltpu.make_async_copy(
                src.at[idx_v.at[c]], rows_v.at[b], sems[b])

        def out_copy(c, b):
            return pltpu.make_async_copy(
                out_v.at[b],
                out_hbm.at[pl.ds(wid * APW + c * CHUNK_ATOMS, CHUNK_ATOMS)],
                osems[b])

        for b in range(nbuf - 1):
            copy(b, b).start()

        def quad_body(i, _):
            c0 = i * nbuf
            for b in range(nbuf):
                c = c0 + b

                @pl.when(c + nbuf - 1 < CHUNKS)
                def _():
                    copy(c + nbuf - 1, (b + nbuf - 1) % nbuf).start()

                copy(c, b).wait()

                @pl.when(c >= nbuf)
                def _():
                    out_copy(c - nbuf, b).wait()

                for a in range(CHUNK_ATOMS):
                    if dtype == jnp.bfloat16:
                        accs = []
                        for g in range(groups):
                            lo, hi = plsc.unpack(
                                rows_v[b, a * MAX_NEI, pl.ds(lanes * g, lanes)],
                                format=plsc.PackFormat.INTERLEAVED)
                            accs.append([lo, hi])
                        for r in range(1, MAX_NEI):
                            for g in range(groups):
                                lo, hi = plsc.unpack(
                                    rows_v[b, a * MAX_NEI + r,
                                           pl.ds(lanes * g, lanes)],
                                    format=plsc.PackFormat.INTERLEAVED)
                                accs[g][0] = accs[g][0] + lo
                                accs[g][1] = accs[g][1] + hi
                        for g in range(groups):
                            out_v[b, a, pl.ds(lanes * g, lanes)] = plsc.pack(
                                accs[g][0], accs[g][1],
                                format=plsc.PackFormat.INTERLEAVED)
                    else:
                        accs = [rows_v[b, a * MAX_NEI, pl.ds(lanes * g, lanes)]
                                for g in range(groups)]
                        for r in range(1, MAX_NEI):
                            for g in range(groups):
                                accs[g] = accs[g] + rows_v[
                                    b, a * MAX_NEI + r, pl.ds(lanes * g, lanes)]
                        for g in range(groups):
                            out_v[b, a, pl.ds(lanes * g, lanes)] = accs[g]
                out_copy(c, b).start()
            return 0

        lax.fori_loop(0, CHUNKS // nbuf, quad_body, 0)
        for b in range(nbuf):
            out_copy(CHUNKS - nbuf + b, b).wait()

    return gsum


_gsum_msg = _make_sc_gather_sum(N_ATOMS, HIDDEN, jnp.bfloat16, True)
_gsum_bond = _make_sc_gather_sum(320000, 16, jnp.float32, False)


_ROWS = 2000
_GRID = N_ATOMS // _ROWS


def _row_mask(pid, x):
    rows = lax.broadcasted_iota(jnp.int32, x.shape, 0) + pid * _ROWS
    return jnp.where(rows == 0, 0.0, x)


def _prologue_body(x_ref, wi_ref, w0_ref, w1_ref, w2_ref,
                   inp_ref, inpb_ref, h_ref):
    pid = pl.program_id(0)
    x = x_ref[...]
    inp = jnp.dot(x, wi_ref[...], preferred_element_type=jnp.float32)
    inp = _row_mask(pid, inp)
    inp_ref[...] = inp
    inpb_ref[...] = inp.astype(jnp.bfloat16)
    h = _gelu_exact(jnp.dot(x, w0_ref[...], preferred_element_type=jnp.float32))
    h = _gelu_exact(jnp.dot(h, w1_ref[...], preferred_element_type=jnp.float32))
    h = _gelu_exact(jnp.dot(h, w2_ref[...], preferred_element_type=jnp.float32))
    h_ref[...] = h


def _tc_prologue(f_atoms, W_i, W0, W1, W2):
    return pl.pallas_call(
        _prologue_body,
        grid=(_GRID,),
        in_specs=[
            pl.BlockSpec((_ROWS, ATOM_FDIM), lambda i: (i, 0)),
            pl.BlockSpec((ATOM_FDIM, HIDDEN), lambda i: (0, 0)),
            pl.BlockSpec((ATOM_FDIM, HIDDEN), lambda i: (0, 0)),
            pl.BlockSpec((HIDDEN, HIDDEN), lambda i: (0, 0)),
            pl.BlockSpec((HIDDEN, HIDDEN), lambda i: (0, 0)),
        ],
        out_specs=[
            pl.BlockSpec((_ROWS, HIDDEN), lambda i: (i, 0)),
            pl.BlockSpec((_ROWS, HIDDEN), lambda i: (i, 0)),
            pl.BlockSpec((_ROWS, HIDDEN), lambda i: (i, 0)),
        ],
        out_shape=[
            jax.ShapeDtypeStruct((N_ATOMS, HIDDEN), jnp.float32),
            jax.ShapeDtypeStruct((N_ATOMS, HIDDEN), jnp.bfloat16),
            jax.ShapeDtypeStruct((N_ATOMS, HIDDEN), jnp.float32),
        ],
    )(f_atoms, W_i, W0, W1, W2)


def _update_body(m_ref, s_ref, b_ref, wt_ref, wb_ref, o_ref, ob_ref):
    pid = pl.program_id(0)
    s = s_ref[...].astype(jnp.float32)
    m = (m_ref[...]
         + jnp.dot(s, wt_ref[...], preferred_element_type=jnp.float32)
         + jnp.dot(b_ref[...], wb_ref[...], preferred_element_type=jnp.float32))
    m = _row_mask(pid, m)
    o_ref[...] = m
    ob_ref[...] = m.astype(jnp.bfloat16)


def _tc_update(message, s, sumb, Wh_top, Wh_bot16):
    return pl.pallas_call(
        _update_body,
        grid=(_GRID,),
        in_specs=[
            pl.BlockSpec((_ROWS, HIDDEN), lambda i: (i, 0)),
            pl.BlockSpec((_ROWS, HIDDEN), lambda i: (i, 0)),
            pl.BlockSpec((_ROWS, 16), lambda i: (i, 0)),
            pl.BlockSpec((HIDDEN, HIDDEN), lambda i: (0, 0)),
            pl.BlockSpec((16, HIDDEN), lambda i: (0, 0)),
        ],
        out_specs=[
            pl.BlockSpec((_ROWS, HIDDEN), lambda i: (i, 0)),
            pl.BlockSpec((_ROWS, HIDDEN), lambda i: (i, 0)),
        ],
        out_shape=[
            jax.ShapeDtypeStruct((N_ATOMS, HIDDEN), jnp.float32),
            jax.ShapeDtypeStruct((N_ATOMS, HIDDEN), jnp.bfloat16),
        ],
    )(message, s, sumb, Wh_top, Wh_bot16)


def _final_body(h_ref, s_ref, wt_ref, wb_ref, o_ref):
    s = s_ref[...].astype(jnp.float32)
    o = (jnp.dot(h_ref[...], wt_ref[...], preferred_element_type=jnp.float32)
         + jnp.dot(s, wb_ref[...], preferred_element_type=jnp.float32))
    o_ref[...] = _gelu_exact(o)


def _tc_final(h, s, Wo_top, Wo_bot):
    return pl.pallas_call(
        _final_body,
        grid=(_GRID,),
        in_specs=[
            pl.BlockSpec((_ROWS, HIDDEN), lambda i: (i, 0)),
            pl.BlockSpec((_ROWS, HIDDEN), lambda i: (i, 0)),
            pl.BlockSpec((HIDDEN, HIDDEN), lambda i: (0, 0)),
            pl.BlockSpec((HIDDEN, HIDDEN), lambda i: (0, 0)),
        ],
        out_specs=pl.BlockSpec((_ROWS, HIDDEN), lambda i: (i, 0)),
        out_shape=jax.ShapeDtypeStruct((N_ATOMS, HIDDEN), jnp.float32),
    )(h, s, Wo_top, Wo_bot)


def _pack_idx(idx):
    idx = jnp.pad(idx.astype(jnp.int32), ((0, N_PAD - N_ATOMS), (0, 0)))
    return idx.reshape(NW, CHUNKS, 128)


def kernel(f_atoms, f_bonds, a2a, a2b, W_i, W_ah0, W_ah1, W_ah2,
           W_h0, W_h1, W_h2, W_o):
    idx_a = _pack_idx(a2a)
    idx_b = _pack_idx(a2b)
    f_bonds16 = jnp.pad(f_bonds, ((0, 0), (0, 16 - BOND_FDIM)))

    W_h = [W_h0, W_h1, W_h2]
    Wh_top = [w[:HIDDEN] for w in W_h]
    Wh_bot16 = [jnp.pad(w[HIDDEN:], ((0, 2), (0, 0))) for w in W_h]

    inp, inp_bf, h = _tc_prologue(f_atoms, W_i, W_ah0, W_ah1, W_ah2)
    sumb = _gsum_bond(f_bonds16, idx_b)[:N_ATOMS]

    message, message_bf = inp, inp_bf
    for d in range(DEPTH):
        s = _gsum_msg(message_bf, idx_a)[:N_ATOMS]
        message, message_bf = _tc_update(message, s, sumb,
                                         Wh_top[d], Wh_bot16[d])

    s = _gsum_msg(message_bf, idx_a)[:N_ATOMS]
    return _tc_final(h, s, W_o[:HIDDEN], W_o[HIDDEN:])
